# Initial kernel scaffold; baseline (speedup 1.0000x reference)
import jax
import jax.numpy as jnp
from jax.experimental import pallas as pl


def kernel(x_img, Wg, W1, b1, W2, b2):
    raise NotImplementedError("write your pallas kernel here")


# speedup vs baseline: 2.7740x; 2.7740x over previous
import functools

import jax
import jax.numpy as jnp
from jax.experimental import pallas as pl
from jax.experimental.pallas import tpu as pltpu

ROW_TILE = 256


def _router_body(x_ref, wg_ref, e_ref, w_ref):
    x = x_ref[...]
    logits = jax.lax.dot_general(
        x, wg_ref[...], (((1,), (1,)), ((), ())),
        preferred_element_type=jnp.float32)
    m = jnp.max(logits, axis=-1, keepdims=True)
    ex = jnp.exp(logits - m)
    p = ex / jnp.sum(ex, axis=-1, keepdims=True)
    num_e = p.shape[-1]
    iota = jax.lax.broadcasted_iota(jnp.int32, p.shape, 1)
    p1 = jnp.max(p, axis=-1, keepdims=True)
    i1 = jnp.min(jnp.where(p == p1, iota, num_e), axis=-1, keepdims=True)
    pm = jnp.where(iota == i1, -jnp.inf, p)
    p2 = jnp.max(pm, axis=-1, keepdims=True)
    i2 = jnp.min(jnp.where(pm == p2, iota, num_e), axis=-1, keepdims=True)
    wsum = p1 + p2
    e_ref[...] = jnp.concatenate([i1, i2], axis=1).astype(jnp.int32)
    w_ref[...] = jnp.concatenate([p1 / wsum, p2 / wsum], axis=1)


def _route(x2d, wg):
    n = x2d.shape[0]
    return pl.pallas_call(
        _router_body,
        out_shape=(
            jax.ShapeDtypeStruct((n, 2), jnp.int32),
            jax.ShapeDtypeStruct((n, 2), jnp.float32),
        ),
    )(x2d, wg)


def _dispatch_meta(eidx, num_e, n, row_tile, num_tiles):
    p = 2 * n
    pad = num_tiles * row_tile
    e_flat = jnp.concatenate([eidx[:, 0], eidx[:, 1]])
    sizes = jnp.bincount(e_flat, length=num_e).astype(jnp.int32)
    offs = jnp.cumsum(sizes) - sizes
    perm = jnp.argsort(e_flat, stable=True).astype(jnp.int32)
    pad_sizes = ((sizes + row_tile - 1) // row_tile) * row_tile
    pad_offs = jnp.cumsum(pad_sizes) - pad_sizes
    pad_end = pad_offs + pad_sizes
    tile_start = jnp.arange(num_tiles, dtype=jnp.int32) * row_tile
    be = jnp.clip(jnp.searchsorted(pad_end, tile_start, side='right'),
                  0, num_e - 1).astype(jnp.int32)
    vt = (tile_start < pad_end[num_e - 1]).astype(jnp.int32)
    s = jnp.arange(pad, dtype=jnp.int32)
    g = jnp.clip(jnp.searchsorted(pad_end, s, side='right'),
                 0, num_e - 1).astype(jnp.int32)
    j = s - pad_offs[g]
    valid = j < sizes[g]
    spos = jnp.where(valid, offs[g] + j, 0)
    pairs = perm[spos]
    src_tok = jnp.where(valid, pairs % n, 0).astype(jnp.int32)
    gp = e_flat[perm]
    pos = jnp.arange(p, dtype=jnp.int32)
    slot_sorted = pad_offs[gp] + (pos - offs[gp])
    slot_of_pair = jnp.zeros((p,), jnp.int32).at[perm].set(
        slot_sorted.astype(jnp.int32))
    return be, vt, src_tok, slot_of_pair[:n], slot_of_pair[n:]


def _gelu_exact(x):
    return 0.5 * x * (1.0 + jax.lax.erf(x * (2.0 ** -0.5)))


def _mlp1_body(be_ref, vt_ref, x_ref, w1_ref, b1_ref, h_ref):
    i = pl.program_id(0)

    @pl.when(vt_ref[i] == 1)
    def _():
        acc = jnp.dot(x_ref[...], w1_ref[0],
                      preferred_element_type=jnp.float32)
        acc = acc + b1_ref[0]
        h_ref[...] = _gelu_exact(acc)


def _mlp2_body(be_ref, vt_ref, h_ref, w2_ref, b2_ref, y_ref):
    i = pl.program_id(0)

    @pl.when(vt_ref[i] == 1)
    def _():
        acc = jnp.dot(h_ref[...], w2_ref[0],
                      preferred_element_type=jnp.float32)
        y_ref[...] = acc + b2_ref[0]


def _grouped_mlp(x_pad, w1, b1, w2, b2, be, vt, num_tiles, row_tile):
    e, d, c = w1.shape
    b1r = b1.reshape(e, 1, c)
    b2r = b2.reshape(e, 1, b2.shape[-1])
    pad = num_tiles * row_tile

    grid1 = pltpu.PrefetchScalarGridSpec(
        num_scalar_prefetch=2,
        grid=(num_tiles,),
        in_specs=[
            pl.BlockSpec((row_tile, d), lambda i, be, vt: (i, 0)),
            pl.BlockSpec((1, d, c), lambda i, be, vt: (be[i], 0, 0)),
            pl.BlockSpec((1, 1, c), lambda i, be, vt: (be[i], 0, 0)),
        ],
        out_specs=pl.BlockSpec((row_tile, c), lambda i, be, vt: (i, 0)),
    )
    h_pad = pl.pallas_call(
        _mlp1_body,
        grid_spec=grid1,
        out_shape=jax.ShapeDtypeStruct((pad, c), jnp.float32),
    )(be, vt, x_pad, w1, b1r)

    c2 = w2.shape[-1]
    grid2 = pltpu.PrefetchScalarGridSpec(
        num_scalar_prefetch=2,
        grid=(num_tiles,),
        in_specs=[
            pl.BlockSpec((row_tile, c), lambda i, be, vt: (i, 0)),
            pl.BlockSpec((1, c, c2), lambda i, be, vt: (be[i], 0, 0)),
            pl.BlockSpec((1, 1, c2), lambda i, be, vt: (be[i], 0, 0)),
        ],
        out_specs=pl.BlockSpec((row_tile, c2), lambda i, be, vt: (i, 0)),
    )
    return pl.pallas_call(
        _mlp2_body,
        grid_spec=grid2,
        out_shape=jax.ShapeDtypeStruct((pad, c2), jnp.float32),
    )(be, vt, h_pad, w2, b2r)


def kernel(x_img, Wg, W1, b1, W2, b2):
    b, s, d = x_img.shape
    e, _, c = W1.shape
    n = b * s
    x2d = x_img.reshape(n, d)

    eidx, w = _route(x2d, Wg)

    num_tiles = (2 * n) // ROW_TILE + e
    be, vt, src_tok, slot0, slot1 = _dispatch_meta(
        eidx, e, n, ROW_TILE, num_tiles)

    x_pad = x2d[src_tok]
    y_pad = _grouped_mlp(x_pad, W1, b1, W2, b2, be, vt, num_tiles, ROW_TILE)
    out = w[:, 0:1] * y_pad[slot0] + w[:, 1:2] * y_pad[slot1]
    return out.reshape(b, s, c)
